# baseline (speedup 1.0000x reference)
import functools

import jax
import jax.numpy as jnp
from jax import lax
from jax.experimental import pallas as pl
from jax.experimental.pallas import tpu as pltpu
from jax.experimental.pallas import tpu_sc as plsc

NC = 2
NS = 16
NW = NC * NS
L = 16

E = 320000
N = 10000
D = 128
EPW = E // NW
K = 64
NCH = EPW // K
KT = EPW - NCH * K


def _edge_kernel_body(pos_hbm, interp_hbm, ei_hbm,
                      feat_hbm, dist_hbm,
                      pos_v, sidx0, sidx1, didx0, didx1,
                      ibuf0, ibuf1, dbuf0, dbuf1, interp_sh,
                      sx0, sx1, ax0, ax1, ig0, ig1, io0, io1, do0, do1,
                      stg):
  sid = lax.axis_index("s")
  wid = sid * NC + lax.axis_index("c")
  base = wid * EPW

  rows_per_sub = 624

  def stg_cp():
    off = pl.multiple_of(sid * rows_per_sub, 8)
    return pltpu.make_async_copy(
        interp_hbm.at[pl.ds(off, rows_per_sub)],
        interp_sh.at[pl.ds(off, rows_per_sub)], stg)

  def stg_last_cp():
    off = (NS - 1) * rows_per_sub
    return pltpu.make_async_copy(
        interp_hbm.at[pl.ds(off, N - off)],
        interp_sh.at[pl.ds(off, N - off)], stg)

  @pl.when(sid < NS - 1)
  def _():
    stg_cp().start()

  @pl.when(sid == NS - 1)
  def _():
    stg_last_cp().start()

  pltpu.sync_copy(pos_hbm, pos_v)

  sidx = (sidx0, sidx1)
  didx = (didx0, didx1)
  ibufs = (ibuf0, ibuf1)
  dbufs = (dbuf0, dbuf1)
  sxs = (sx0, sx1)
  axs = (ax0, ax1)
  igs = (ig0, ig1)
  ios = (io0, io1)
  dos = (do0, do1)

  def sx_cp(c, b, n=K):
    return pltpu.make_async_copy(
        ei_hbm.at[pl.ds(base + c * K, n)], sidx[b].at[pl.ds(0, n)],
        sxs[b])

  def ax_cp(c, b, n=K):
    return pltpu.make_async_copy(
        ei_hbm.at[pl.ds(E + base + c * K, n)], didx[b].at[pl.ds(0, n)],
        axs[b])

  def ig_cp(c, b, n=K, hbm=False):
    del c
    table = interp_hbm if hbm else interp_sh
    return pltpu.make_async_copy(
        table.at[sidx[b].at[pl.ds(0, n)]],
        ibufs[b].at[pl.ds(0, n)], igs[b])

  def io_cp(c, b, n=K):
    return pltpu.make_async_copy(
        ibufs[b].at[pl.ds(0, n)],
        feat_hbm.at[pl.ds(base + c * K, n)], ios[b])

  def do_cp(c, b, n=K):
    return pltpu.make_async_copy(
        dbufs[b].at[pl.ds(0, n)],
        dist_hbm.at[pl.ds(base + c * K, n)], dos[b])

  def chunk(c, b, first, last, n=K, n_prev=K, n_next=K, hbm=False):
    sx_cp(c, b, n).wait()
    ax_cp(c, b, n).wait()
    if not last:
      sx_cp(c + 1, 1 - b, n_next).start()
      ax_cp(c + 1, 1 - b, n_next).start()
    if not first:
      io_cp(c - 2, b, n_prev).wait()
    ig_cp(c, b, n, hbm).start()
    if not first:
      do_cp(c - 2, b, n_prev).wait()
    for t in range(n // L):
      si = sidx[b][pl.ds(t * L, L)] * 3
      di = didx[b][pl.ds(t * L, L)] * 3
      acc = jnp.full((L,), 1e-6, jnp.float32)
      for j in range(3):
        a = plsc.load_gather(pos_v, [si + j])
        d = plsc.load_gather(pos_v, [di + j])
        dd = a - d
        acc = acc + dd * dd
      dbufs[b][pl.ds(t * L, L)] = 1.0 / acc
    do_cp(c, b, n).start()
    ig_cp(c, b, n, hbm).wait()
    io_cp(c, b, n).start()
    return 0

  HB = 16
  sx_cp(0, 0).start()
  ax_cp(0, 0).start()
  chunk(0, 0, True, False, hbm=True)
  chunk(1, 1, True, False, hbm=True)

  def pair(g, carry, hbm=False):
    c0 = 2 * g
    chunk(c0, 0, False, False, hbm=hbm)
    chunk(c0 + 1, 1, False, False, hbm=hbm)
    return carry

  lax.fori_loop(1, HB // 2, functools.partial(pair, hbm=True), 0)
  @pl.when(sid < NS - 1)
  def _():
    stg_cp().wait()

  @pl.when(sid == NS - 1)
  def _():
    stg_last_cp().wait()

  plsc.subcore_barrier()
  lax.fori_loop(HB // 2, NCH // 2 - 1, pair, 0)
  chunk(NCH - 2, 0, False, False)
  chunk(NCH - 1, 1, False, False, n_next=KT)
  chunk(NCH, 0, False, True, n=KT)
  do_cp(NCH - 1, 1).wait()
  io_cp(NCH - 1, 1).wait()
  do_cp(NCH, 0, KT).wait()
  io_cp(NCH, 0, KT).wait()


@jax.jit
def _run(pos, interp, ei):
  kern = pl.kernel(
      _edge_kernel_body,
      out_type=(
          jax.ShapeDtypeStruct((E, D), jnp.float32),
          jax.ShapeDtypeStruct((E,), jnp.float32),
      ),
      mesh=plsc.VectorSubcoreMesh(
          core_axis_name="c", subcore_axis_name="s",
          num_cores=NC, num_subcores=NS),
      compiler_params=pltpu.CompilerParams(needs_layout_passes=False),
      scratch_types=[
          pltpu.VMEM((N * 3,), jnp.float32),
          pltpu.VMEM((K,), jnp.int32),
          pltpu.VMEM((K,), jnp.int32),
          pltpu.VMEM((K,), jnp.int32),
          pltpu.VMEM((K,), jnp.int32),
          pltpu.VMEM((K, D), jnp.float32),
          pltpu.VMEM((K, D), jnp.float32),
          pltpu.VMEM((K,), jnp.float32),
          pltpu.VMEM((K,), jnp.float32),
          pltpu.VMEM_SHARED((N, D), jnp.float32),
          pltpu.SemaphoreType.DMA,
          pltpu.SemaphoreType.DMA,
          pltpu.SemaphoreType.DMA,
          pltpu.SemaphoreType.DMA,
          pltpu.SemaphoreType.DMA,
          pltpu.SemaphoreType.DMA,
          pltpu.SemaphoreType.DMA,
          pltpu.SemaphoreType.DMA,
          pltpu.SemaphoreType.DMA,
          pltpu.SemaphoreType.DMA,
          pltpu.SemaphoreType.DMA,
      ],
  )
  return kern(pos, interp, ei)


def kernel(pos, interp, edge_index):
  feat, dist = _run(pos.reshape(-1), interp,
                    edge_index.astype(jnp.int32).reshape(-1))
  return (feat, dist)


# speedup vs baseline: 1.0927x; 1.0927x over previous
import functools

import jax
import jax.numpy as jnp
from jax import lax
from jax.experimental import pallas as pl
from jax.experimental.pallas import tpu as pltpu
from jax.experimental.pallas import tpu_sc as plsc

NC = 2
NS = 16
NW = NC * NS
L = 16

E = 320000
N = 10000
D = 128
EPW = E // NW
K = 64
NCH = EPW // K
KT = EPW - NCH * K


def _edge_kernel_body(pos_hbm, interp_hbm, ei_hbm,
                      feat_hbm, dist_hbm,
                      pos_v, sidx0, sidx1, didx0, didx1,
                      ibuf0, ibuf1, dbuf0, dbuf1, interp_sh,
                      sx0, sx1, ax0, ax1, ig0, ig1, io0, io1, do0, do1):
  sid = lax.axis_index("s")
  wid = sid * NC + lax.axis_index("c")
  base = wid * EPW

  rows_per_sub = 624

  @pl.when(sid < NS - 1)
  def _():
    off = pl.multiple_of(sid * rows_per_sub, 8)
    pltpu.sync_copy(interp_hbm.at[pl.ds(off, rows_per_sub)],
                    interp_sh.at[pl.ds(off, rows_per_sub)])

  @pl.when(sid == NS - 1)
  def _():
    off = (NS - 1) * rows_per_sub
    pltpu.sync_copy(interp_hbm.at[pl.ds(off, N - off)],
                    interp_sh.at[pl.ds(off, N - off)])

  pltpu.sync_copy(pos_hbm, pos_v)
  plsc.subcore_barrier()

  sidx = (sidx0, sidx1)
  didx = (didx0, didx1)
  ibufs = (ibuf0, ibuf1)
  dbufs = (dbuf0, dbuf1)
  sxs = (sx0, sx1)
  axs = (ax0, ax1)
  igs = (ig0, ig1)
  ios = (io0, io1)
  dos = (do0, do1)

  def sx_cp(c, b, n=K):
    return pltpu.make_async_copy(
        ei_hbm.at[pl.ds(base + c * K, n)], sidx[b].at[pl.ds(0, n)],
        sxs[b])

  def ax_cp(c, b, n=K):
    return pltpu.make_async_copy(
        ei_hbm.at[pl.ds(E + base + c * K, n)], didx[b].at[pl.ds(0, n)],
        axs[b])

  def ig_cp(c, b, n=K):
    del c
    return pltpu.make_async_copy(
        interp_sh.at[sidx[b].at[pl.ds(0, n)]],
        ibufs[b].at[pl.ds(0, n)], igs[b])

  def io_cp(c, b, n=K):
    return pltpu.make_async_copy(
        ibufs[b].at[pl.ds(0, n)],
        feat_hbm.at[pl.ds(base + c * K, n)], ios[b])

  def do_cp(c, b, n=K):
    return pltpu.make_async_copy(
        dbufs[b].at[pl.ds(0, n)],
        dist_hbm.at[pl.ds(base + c * K, n)], dos[b])

  def chunk(c, b, first, last, n=K, n_prev=K, n_next=K):
    sx_cp(c, b, n).wait()
    ax_cp(c, b, n).wait()
    if not last:
      sx_cp(c + 1, 1 - b, n_next).start()
      ax_cp(c + 1, 1 - b, n_next).start()
    if not first:
      io_cp(c - 2, b, n_prev).wait()
    ig_cp(c, b, n).start()
    if not first:
      do_cp(c - 2, b, n_prev).wait()
    for t in range(n // L):
      si = sidx[b][pl.ds(t * L, L)] * 3
      di = didx[b][pl.ds(t * L, L)] * 3
      acc = jnp.full((L,), 1e-6, jnp.float32)
      for j in range(3):
        a = plsc.load_gather(pos_v, [si + j])
        d = plsc.load_gather(pos_v, [di + j])
        dd = a - d
        acc = acc + dd * dd
      dbufs[b][pl.ds(t * L, L)] = 1.0 / acc
    do_cp(c, b, n).start()
    ig_cp(c, b, n).wait()
    io_cp(c, b, n).start()
    return 0

  sx_cp(0, 0).start()
  ax_cp(0, 0).start()
  chunk(0, 0, True, False)
  chunk(1, 1, True, False)

  def pair(g, carry):
    c0 = 2 * g
    chunk(c0, 0, False, False)
    chunk(c0 + 1, 1, False, False)
    return carry

  lax.fori_loop(1, NCH // 2 - 1, pair, 0)
  chunk(NCH - 2, 0, False, False)
  chunk(NCH - 1, 1, False, False, n_next=KT)
  chunk(NCH, 0, False, True, n=KT)
  do_cp(NCH - 1, 1).wait()
  io_cp(NCH - 1, 1).wait()
  do_cp(NCH, 0, KT).wait()
  io_cp(NCH, 0, KT).wait()


@jax.jit
def _run(pos, interp, ei):
  kern = pl.kernel(
      _edge_kernel_body,
      out_type=(
          jax.ShapeDtypeStruct((E, D), jnp.float32),
          jax.ShapeDtypeStruct((E,), jnp.float32),
      ),
      mesh=plsc.VectorSubcoreMesh(
          core_axis_name="c", subcore_axis_name="s",
          num_cores=NC, num_subcores=NS),
      compiler_params=pltpu.CompilerParams(needs_layout_passes=False),
      scratch_types=[
          pltpu.VMEM((N * 3,), jnp.float32),
          pltpu.VMEM((K,), jnp.int32),
          pltpu.VMEM((K,), jnp.int32),
          pltpu.VMEM((K,), jnp.int32),
          pltpu.VMEM((K,), jnp.int32),
          pltpu.VMEM((K, D), jnp.float32),
          pltpu.VMEM((K, D), jnp.float32),
          pltpu.VMEM((K,), jnp.float32),
          pltpu.VMEM((K,), jnp.float32),
          pltpu.VMEM_SHARED((N, D), jnp.float32),
          pltpu.SemaphoreType.DMA,
          pltpu.SemaphoreType.DMA,
          pltpu.SemaphoreType.DMA,
          pltpu.SemaphoreType.DMA,
          pltpu.SemaphoreType.DMA,
          pltpu.SemaphoreType.DMA,
          pltpu.SemaphoreType.DMA,
          pltpu.SemaphoreType.DMA,
          pltpu.SemaphoreType.DMA,
          pltpu.SemaphoreType.DMA,
      ],
  )
  return kern(pos, interp, ei)


def kernel(pos, interp, edge_index):
  feat, dist = _run(pos.reshape(-1), interp,
                    edge_index.astype(jnp.int32).reshape(-1))
  return (feat, dist)
